# baseline (speedup 1.0000x reference)
from functools import partial

import numpy as np
import jax
import jax.numpy as jnp
from jax import lax
from jax.experimental import pallas as pl
from jax.experimental.pallas import tpu as pltpu

_EPS = 1e-5
_INV_SQRT2 = 0.7071067811865475


def _gelu_exact(v):
    return 0.5 * v * (1.0 + lax.erf(v * _INV_SQRT2))


def _combine_dx(u, b, p2):
    return (u[1] + pltpu.roll(u[2], p2 - 1, axis=1)
            + pltpu.roll(u[0], 1, axis=1) + b)


def _stage1(x_ref, skip3_ref, mup_ref, w1u_ref, w1s_ref, b1_ref, ws_ref,
            wss_ref, bs_ref, mv_ref, y1_ref, s1_ref, q1_ref, ys_ref,
            slab_ref, *, stride, margin, p2):
    c2, p4 = x_ref.shape[1], x_ref.shape[2]
    d = skip3_ref.shape[2]
    m = margin

    slab_ref[:, 0:m] = jnp.zeros((c2, m), jnp.bfloat16)
    slab_ref[:, m + p2:m + p2 + m] = jnp.zeros((c2, m), jnp.bfloat16)

    up = jnp.dot(x_ref[...].reshape(c2, p4), mup_ref[...],
                 preferred_element_type=jnp.float32)
    slab_ref[:, m:m + p2] = up.astype(jnp.bfloat16)

    u = [None, None, None]
    for t, dy in enumerate((-1, 0, 1)):
        su = slab_ref[:, m + dy * stride:m + dy * stride + p2]
        sk = skip3_ref[0, t]
        for j, dx in enumerate((-1, 0, 1)):
            tap = (dy + 1) * 3 + (dx + 1)
            term = (jnp.dot(w1u_ref[tap], su,
                            preferred_element_type=jnp.float32)
                    + jnp.dot(w1s_ref[tap], sk,
                              preferred_element_type=jnp.float32))
            u[j] = term if u[j] is None else u[j] + term
    raw = _combine_dx(u, b1_ref[...], p2)

    ys = (jnp.dot(ws_ref[...], slab_ref[:, m:m + p2],
                  preferred_element_type=jnp.float32)
          + jnp.dot(wss_ref[...], skip3_ref[0, 1],
                    preferred_element_type=jnp.float32) + bs_ref[...])

    y1_ref[...] = raw.reshape(1, d, p2).astype(y1_ref.dtype)
    ys_ref[...] = ys.reshape(1, d, p2).astype(ys_ref.dtype)
    mv = mv_ref[...]
    s1_ref[...] = jnp.dot(raw, mv,
                          preferred_element_type=jnp.float32).reshape(1, d, 1)
    q1_ref[...] = jnp.dot(raw * raw, mv,
                          preferred_element_type=jnp.float32).reshape(1, d, 1)


def _stage2(y1_ref, sc1_ref, sh1_ref, gm_ref, w2_ref, b2_ref, mv_ref,
            y2_ref, s2_ref, q2_ref, slab_ref, *, stride, margin, p2):
    d = y1_ref.shape[1]
    m = margin

    slab_ref[:, 0:m] = jnp.zeros((d, m), jnp.bfloat16)
    slab_ref[:, m + p2:m + p2 + m] = jnp.zeros((d, m), jnp.bfloat16)

    act = _gelu_exact(y1_ref[...].reshape(d, p2).astype(jnp.float32)
                      * sc1_ref[...] + sh1_ref[...])
    slab_ref[:, m:m + p2] = act.astype(jnp.bfloat16) * gm_ref[...]

    u = [None, None, None]
    for t, dy in enumerate((-1, 0, 1)):
        sa = slab_ref[:, m + dy * stride:m + dy * stride + p2]
        for j, dx in enumerate((-1, 0, 1)):
            tap = (dy + 1) * 3 + (dx + 1)
            term = jnp.dot(w2_ref[tap], sa,
                           preferred_element_type=jnp.float32)
            u[j] = term if u[j] is None else u[j] + term
    raw = _combine_dx(u, b2_ref[...], p2)

    y2_ref[...] = raw.reshape(1, d, p2).astype(y2_ref.dtype)
    mv = mv_ref[...]
    s2_ref[...] = jnp.dot(raw, mv,
                          preferred_element_type=jnp.float32).reshape(1, d, 1)
    q2_ref[...] = jnp.dot(raw * raw, mv,
                          preferred_element_type=jnp.float32).reshape(1, d, 1)


def _stage3(y2_ref, sc2_ref, sh2_ref, ys_ref, out_ref):
    d, p2 = y2_ref.shape[1], y2_ref.shape[2]
    act = _gelu_exact(y2_ref[...].reshape(d, p2).astype(jnp.float32)
                      * sc2_ref[...] + sh2_ref[...])
    out_ref[...] = (act + ys_ref[...].reshape(d, p2).astype(jnp.float32)
                    ).reshape(1, d, p2)


def _finalize_bn(s, q, gamma, beta, count):
    tot = jnp.sum(s[:, :, 0], axis=0)
    totsq = jnp.sum(q[:, :, 0], axis=0)
    mu = tot / count
    var = totsq / count - mu * mu
    inv = lax.rsqrt(jnp.maximum(var, 0.0) + _EPS)
    sc = gamma * inv
    sh = beta - mu * sc
    d = sc.shape[0]
    return sc.reshape(d, 1), sh.reshape(d, 1)


def _params(sems):
    return pltpu.CompilerParams(dimension_semantics=sems,
                                vmem_limit_bytes=100 * 1024 * 1024)


def kernel(x, skip, w1, b1, g1, be1, w2, b2, g2, be2, wsx, wss, bs):
    n, c2, hh, ww = x.shape
    _, d, hgt, wid = skip.shape
    p4 = hh * ww
    stride = wid + 2
    p2 = hgt * stride
    m = max(128, pl.cdiv(stride + 1, 128) * 128)
    slen = 2 * m + p2
    bf16, f32 = jnp.bfloat16, jnp.float32

    xb = x.reshape(n, c2, p4).astype(bf16)

    skp = jnp.pad(skip.astype(bf16), ((0, 0), (0, 0), (0, 0), (0, 2))
                  ).reshape(n, d, p2)
    skp = jnp.pad(skp, ((0, 0), (0, 0), (m, m)))
    skip3 = jnp.stack([skp[:, :, m + k * stride:m + k * stride + p2]
                       for k in (-1, 0, 1)], axis=1)

    w1u = w1[:, :, :c2].astype(bf16)
    w1s = w1[:, :, c2:].astype(bf16)
    w2b = w2.astype(bf16)
    wsxb = wsx.astype(bf16)
    wssb = wss.astype(bf16)

    rr = np.arange(p2) // stride
    cc = np.arange(p2) % stride
    interior = cc < wid
    src = np.where(interior, (rr // 2) * ww + np.minimum(cc, wid - 1) // 2, -1)
    mup = jnp.asarray(np.arange(p4)[:, None] == src[None, :], bf16)
    maskv = jnp.asarray(interior[:, None], f32)
    gmask = jnp.asarray(interior[None, :], bf16)

    y1, s1, q1, ys = pl.pallas_call(
        partial(_stage1, stride=stride, margin=m, p2=p2),
        grid=(n,),
        in_specs=[
            pl.BlockSpec((1, c2, p4), lambda i: (i, 0, 0)),
            pl.BlockSpec((1, 3, d, p2), lambda i: (i, 0, 0, 0)),
            pl.BlockSpec((p4, p2), lambda i: (0, 0)),
            pl.BlockSpec((9, d, c2), lambda i: (0, 0, 0)),
            pl.BlockSpec((9, d, d), lambda i: (0, 0, 0)),
            pl.BlockSpec((d, 1), lambda i: (0, 0)),
            pl.BlockSpec((d, c2), lambda i: (0, 0)),
            pl.BlockSpec((d, d), lambda i: (0, 0)),
            pl.BlockSpec((d, 1), lambda i: (0, 0)),
            pl.BlockSpec((p2, 1), lambda i: (0, 0)),
        ],
        out_specs=(
            pl.BlockSpec((1, d, p2), lambda i: (i, 0, 0)),
            pl.BlockSpec((1, d, 1), lambda i: (i, 0, 0)),
            pl.BlockSpec((1, d, 1), lambda i: (i, 0, 0)),
            pl.BlockSpec((1, d, p2), lambda i: (i, 0, 0)),
        ),
        out_shape=(
            jax.ShapeDtypeStruct((n, d, p2), bf16),
            jax.ShapeDtypeStruct((n, d, 1), f32),
            jax.ShapeDtypeStruct((n, d, 1), f32),
            jax.ShapeDtypeStruct((n, d, p2), bf16),
        ),
        scratch_shapes=[pltpu.VMEM((c2, slen), bf16)],
        compiler_params=_params(("parallel",)),
    )(xb, skip3, mup, w1u, w1s, b1, wsxb, wssb, bs, maskv)

    sc1, sh1 = _finalize_bn(s1, q1, g1, be1, float(n * hgt * wid))

    y2, s2, q2 = pl.pallas_call(
        partial(_stage2, stride=stride, margin=m, p2=p2),
        grid=(n,),
        in_specs=[
            pl.BlockSpec((1, d, p2), lambda i: (i, 0, 0)),
            pl.BlockSpec((d, 1), lambda i: (0, 0)),
            pl.BlockSpec((d, 1), lambda i: (0, 0)),
            pl.BlockSpec((1, p2), lambda i: (0, 0)),
            pl.BlockSpec((9, d, d), lambda i: (0, 0, 0)),
            pl.BlockSpec((d, 1), lambda i: (0, 0)),
            pl.BlockSpec((p2, 1), lambda i: (0, 0)),
        ],
        out_specs=(
            pl.BlockSpec((1, d, p2), lambda i: (i, 0, 0)),
            pl.BlockSpec((1, d, 1), lambda i: (i, 0, 0)),
            pl.BlockSpec((1, d, 1), lambda i: (i, 0, 0)),
        ),
        out_shape=(
            jax.ShapeDtypeStruct((n, d, p2), bf16),
            jax.ShapeDtypeStruct((n, d, 1), f32),
            jax.ShapeDtypeStruct((n, d, 1), f32),
        ),
        scratch_shapes=[pltpu.VMEM((d, slen), bf16)],
        compiler_params=_params(("parallel",)),
    )(y1, sc1, sh1, gmask, w2b, b2, maskv)

    sc2, sh2 = _finalize_bn(s2, q2, g2, be2, float(n * hgt * wid))

    out = pl.pallas_call(
        _stage3,
        grid=(n,),
        in_specs=[
            pl.BlockSpec((1, d, p2), lambda i: (i, 0, 0)),
            pl.BlockSpec((d, 1), lambda i: (0, 0)),
            pl.BlockSpec((d, 1), lambda i: (0, 0)),
            pl.BlockSpec((1, d, p2), lambda i: (i, 0, 0)),
        ],
        out_specs=pl.BlockSpec((1, d, p2), lambda i: (i, 0, 0)),
        out_shape=jax.ShapeDtypeStruct((n, d, p2), f32),
        compiler_params=_params(("parallel",)),
    )(y2, sc2, sh2, ys)

    return out.reshape(n, d, hgt, stride)[:, :, :, :wid]


# speedup vs baseline: 1.4449x; 1.4449x over previous
from functools import partial

import numpy as np
import jax
import jax.numpy as jnp
from jax import lax
from jax.experimental import pallas as pl
from jax.experimental.pallas import tpu as pltpu

_EPS = 1e-5
_INV_SQRT2 = 0.7071067811865475


def _gelu_exact(v):
    return 0.5 * v * (1.0 + lax.erf(v * _INV_SQRT2))


def _combine_dx(u, b, p2):
    return (u[1] + pltpu.roll(u[2], p2 - 1, axis=1)
            + pltpu.roll(u[0], 1, axis=1) + b)


def _conv_on_slab(slab_ref, w_ref, b, mv, *, stride, margin, p2, nrow):
    m = margin
    u = [None, None, None]
    for dy in (-1, 0, 1):
        sl = slab_ref[:, m + dy * stride:m + dy * stride + p2]
        for j, dx in enumerate((-1, 0, 1)):
            tap = (dy + 1) * 3 + (dx + 1)
            term = jnp.dot(w_ref[tap], sl, preferred_element_type=jnp.float32)
            u[j] = term if u[j] is None else u[j] + term
    raw = _combine_dx(u, b, p2)
    s = jnp.dot(raw, mv, preferred_element_type=jnp.float32)
    q = jnp.dot(raw * raw, mv, preferred_element_type=jnp.float32)
    return raw, s.reshape(1, nrow, 1), q.reshape(1, nrow, 1)


def _stage1(x_ref, skip_ref, mup_ref, w1_ref, b1_ref, ws_ref, bs_ref,
            mv_ref, y1_ref, s1_ref, q1_ref, ys_ref, slab_ref,
            *, stride, margin, p2, nrows, width):
    c2, p4 = x_ref.shape[1], x_ref.shape[2]
    d = skip_ref.shape[1]
    c3, m = c2 + d, margin

    slab_ref[:, 0:m] = jnp.zeros((c3, m), jnp.bfloat16)
    slab_ref[:, m + p2:m + p2 + m] = jnp.zeros((c3, m), jnp.bfloat16)

    up = jnp.dot(x_ref[...].reshape(c2, p4).astype(jnp.bfloat16), mup_ref[...],
                 preferred_element_type=jnp.float32)
    slab_ref[0:c2, m:m + p2] = up.astype(jnp.bfloat16)

    sk = skip_ref[...].reshape(d, nrows * width).astype(jnp.bfloat16)
    zg = jnp.zeros((d, 2), jnp.bfloat16)
    for r in range(nrows):
        slab_ref[c2:c3, m + r * stride:m + r * stride + width] = (
            sk[:, r * width:(r + 1) * width])
        slab_ref[c2:c3, m + r * stride + width:m + (r + 1) * stride] = zg

    raw, s, q = _conv_on_slab(slab_ref, w1_ref, b1_ref[...], mv_ref[...],
                              stride=stride, margin=m, p2=p2, nrow=d)

    ys = jnp.dot(ws_ref[...], slab_ref[:, m:m + p2],
                 preferred_element_type=jnp.float32) + bs_ref[...]

    y1_ref[...] = raw.reshape(1, d, p2).astype(y1_ref.dtype)
    ys_ref[...] = ys.reshape(1, d, p2).astype(ys_ref.dtype)
    s1_ref[...] = s
    q1_ref[...] = q


def _stage2(y1_ref, sc1_ref, sh1_ref, gm_ref, w2_ref, b2_ref, mv_ref,
            y2_ref, s2_ref, q2_ref, slab_ref, *, stride, margin, p2):
    d = y1_ref.shape[1]
    m = margin

    slab_ref[:, 0:m] = jnp.zeros((d, m), jnp.bfloat16)
    slab_ref[:, m + p2:m + p2 + m] = jnp.zeros((d, m), jnp.bfloat16)

    act = _gelu_exact(y1_ref[...].reshape(d, p2).astype(jnp.float32)
                      * sc1_ref[...] + sh1_ref[...])
    slab_ref[:, m:m + p2] = act.astype(jnp.bfloat16) * gm_ref[...]

    raw, s, q = _conv_on_slab(slab_ref, w2_ref, b2_ref[...], mv_ref[...],
                              stride=stride, margin=m, p2=p2, nrow=d)

    y2_ref[...] = raw.reshape(1, d, p2).astype(y2_ref.dtype)
    s2_ref[...] = s
    q2_ref[...] = q


def _stage3(y2_ref, sc2_ref, sh2_ref, ys_ref, out_ref,
            *, stride, p2, nrows, width):
    d = y2_ref.shape[1]
    act = _gelu_exact(y2_ref[...].reshape(d, p2).astype(jnp.float32)
                      * sc2_ref[...] + sh2_ref[...])
    v = act + ys_ref[...].reshape(d, p2).astype(jnp.float32)
    for r in range(nrows):
        out_ref[0, :, r * width:(r + 1) * width] = (
            v[:, r * stride:r * stride + width])


def _finalize_bn(s, q, gamma, beta, count):
    tot = jnp.sum(s[:, :, 0], axis=0)
    totsq = jnp.sum(q[:, :, 0], axis=0)
    mu = tot / count
    var = totsq / count - mu * mu
    inv = lax.rsqrt(jnp.maximum(var, 0.0) + _EPS)
    sc = gamma * inv
    sh = beta - mu * sc
    d = sc.shape[0]
    return sc.reshape(d, 1), sh.reshape(d, 1)


def _params(sems):
    return pltpu.CompilerParams(dimension_semantics=sems,
                                vmem_limit_bytes=100 * 1024 * 1024)


def kernel(x, skip, w1, b1, g1, be1, w2, b2, g2, be2, wsx, wss, bs):
    n, c2, hh, ww = x.shape
    _, d, hgt, wid = skip.shape
    c3 = c2 + d
    p4, p = hh * ww, hgt * wid
    stride = wid + 2
    p2 = hgt * stride
    m = max(128, pl.cdiv(stride + 1, 128) * 128)
    slen = 2 * m + p2
    bf16, f32 = jnp.bfloat16, jnp.float32

    xf = x.reshape(n, c2, p4)
    sf = skip.reshape(n, d, p)
    w1b = w1.astype(bf16)
    w2b = w2.astype(bf16)
    wsb = jnp.concatenate([wsx, wss], axis=1).astype(bf16)

    rr = np.arange(p2) // stride
    cc = np.arange(p2) % stride
    interior = cc < wid
    src = np.where(interior, (rr // 2) * ww + np.minimum(cc, wid - 1) // 2, -1)
    mup = jnp.asarray(np.arange(p4)[:, None] == src[None, :], bf16)
    maskv = jnp.asarray(interior[:, None], f32)
    gmask = jnp.asarray(interior[None, :], bf16)

    y1, s1, q1, ys = pl.pallas_call(
        partial(_stage1, stride=stride, margin=m, p2=p2, nrows=hgt,
                width=wid),
        grid=(n,),
        in_specs=[
            pl.BlockSpec((1, c2, p4), lambda i: (i, 0, 0)),
            pl.BlockSpec((1, d, p), lambda i: (i, 0, 0)),
            pl.BlockSpec((p4, p2), lambda i: (0, 0)),
            pl.BlockSpec((9, d, c3), lambda i: (0, 0, 0)),
            pl.BlockSpec((d, 1), lambda i: (0, 0)),
            pl.BlockSpec((d, c3), lambda i: (0, 0)),
            pl.BlockSpec((d, 1), lambda i: (0, 0)),
            pl.BlockSpec((p2, 1), lambda i: (0, 0)),
        ],
        out_specs=(
            pl.BlockSpec((1, d, p2), lambda i: (i, 0, 0)),
            pl.BlockSpec((1, d, 1), lambda i: (i, 0, 0)),
            pl.BlockSpec((1, d, 1), lambda i: (i, 0, 0)),
            pl.BlockSpec((1, d, p2), lambda i: (i, 0, 0)),
        ),
        out_shape=(
            jax.ShapeDtypeStruct((n, d, p2), bf16),
            jax.ShapeDtypeStruct((n, d, 1), f32),
            jax.ShapeDtypeStruct((n, d, 1), f32),
            jax.ShapeDtypeStruct((n, d, p2), bf16),
        ),
        scratch_shapes=[pltpu.VMEM((c3, slen), bf16)],
        compiler_params=_params(("parallel",)),
    )(xf, sf, mup, w1b, b1, wsb, bs, maskv)

    sc1, sh1 = _finalize_bn(s1, q1, g1, be1, float(n * p))

    y2, s2, q2 = pl.pallas_call(
        partial(_stage2, stride=stride, margin=m, p2=p2),
        grid=(n,),
        in_specs=[
            pl.BlockSpec((1, d, p2), lambda i: (i, 0, 0)),
            pl.BlockSpec((d, 1), lambda i: (0, 0)),
            pl.BlockSpec((d, 1), lambda i: (0, 0)),
            pl.BlockSpec((1, p2), lambda i: (0, 0)),
            pl.BlockSpec((9, d, d), lambda i: (0, 0, 0)),
            pl.BlockSpec((d, 1), lambda i: (0, 0)),
            pl.BlockSpec((p2, 1), lambda i: (0, 0)),
        ],
        out_specs=(
            pl.BlockSpec((1, d, p2), lambda i: (i, 0, 0)),
            pl.BlockSpec((1, d, 1), lambda i: (i, 0, 0)),
            pl.BlockSpec((1, d, 1), lambda i: (i, 0, 0)),
        ),
        out_shape=(
            jax.ShapeDtypeStruct((n, d, p2), bf16),
            jax.ShapeDtypeStruct((n, d, 1), f32),
            jax.ShapeDtypeStruct((n, d, 1), f32),
        ),
        scratch_shapes=[pltpu.VMEM((d, slen), bf16)],
        compiler_params=_params(("parallel",)),
    )(y1, sc1, sh1, gmask, w2b, b2, maskv)

    sc2, sh2 = _finalize_bn(s2, q2, g2, be2, float(n * p))

    out = pl.pallas_call(
        partial(_stage3, stride=stride, p2=p2, nrows=hgt, width=wid),
        grid=(n,),
        in_specs=[
            pl.BlockSpec((1, d, p2), lambda i: (i, 0, 0)),
            pl.BlockSpec((d, 1), lambda i: (0, 0)),
            pl.BlockSpec((d, 1), lambda i: (0, 0)),
            pl.BlockSpec((1, d, p2), lambda i: (i, 0, 0)),
        ],
        out_specs=pl.BlockSpec((1, d, p), lambda i: (i, 0, 0)),
        out_shape=jax.ShapeDtypeStruct((n, d, p), f32),
        compiler_params=_params(("parallel",)),
    )(y2, sc2, sh2, ys)

    return out.reshape(n, d, hgt, wid)


# speedup vs baseline: 1.4456x; 1.0005x over previous
from functools import partial

import numpy as np
import jax
import jax.numpy as jnp
from jax import lax
from jax.experimental import pallas as pl
from jax.experimental.pallas import tpu as pltpu

_EPS = 1e-5
_INV_SQRT2 = 0.7071067811865475


def _gelu_exact(v):
    return 0.5 * v * (1.0 + lax.erf(v * _INV_SQRT2))


def _combine_dx(u, b, p2):
    return (u[1] + pltpu.roll(u[2], p2 - 1, axis=1)
            + pltpu.roll(u[0], 1, axis=1) + b)


def _conv_on_slab(slab_ref, w_ref, b, mv, *, stride, margin, p2, nrow):
    m = margin
    u = [None, None, None]
    for dy in (-1, 0, 1):
        sl = slab_ref[:, m + dy * stride:m + dy * stride + p2]
        for j, dx in enumerate((-1, 0, 1)):
            tap = (dy + 1) * 3 + (dx + 1)
            term = jnp.dot(w_ref[tap], sl, preferred_element_type=jnp.float32)
            u[j] = term if u[j] is None else u[j] + term
    raw = _combine_dx(u, b, p2)
    s = jnp.dot(raw, mv, preferred_element_type=jnp.float32)
    q = jnp.dot(raw * raw, mv, preferred_element_type=jnp.float32)
    return raw, s.reshape(1, nrow, 1), q.reshape(1, nrow, 1)


def _stage1(x_ref, skip_ref, mup_ref, w1_ref, b1_ref, ws_ref, bs_ref,
            mv_ref, y1_ref, s1_ref, q1_ref, ys_ref, slab2_ref,
            *, stride, margin, p2, nrows, width):
    c2, p4 = x_ref.shape[1], x_ref.shape[2]
    d = skip_ref.shape[1]
    c3, m = c2 + d, margin
    slab_ref = slab2_ref.at[pl.program_id(0) % 2]

    slab_ref[:, 0:m] = jnp.zeros((c3, m), jnp.bfloat16)
    slab_ref[:, m + p2:m + p2 + m] = jnp.zeros((c3, m), jnp.bfloat16)

    up = jnp.dot(x_ref[...].reshape(c2, p4).astype(jnp.bfloat16), mup_ref[...],
                 preferred_element_type=jnp.float32)
    slab_ref[0:c2, m:m + p2] = up.astype(jnp.bfloat16)

    sk = skip_ref[...].reshape(d, nrows * width).astype(jnp.bfloat16)
    zg = jnp.zeros((d, 2), jnp.bfloat16)
    for r in range(nrows):
        slab_ref[c2:c3, m + r * stride:m + r * stride + width] = (
            sk[:, r * width:(r + 1) * width])
        slab_ref[c2:c3, m + r * stride + width:m + (r + 1) * stride] = zg

    raw, s, q = _conv_on_slab(slab_ref, w1_ref, b1_ref[...], mv_ref[...],
                              stride=stride, margin=m, p2=p2, nrow=d)

    ys = jnp.dot(ws_ref[...], slab_ref[:, m:m + p2],
                 preferred_element_type=jnp.float32) + bs_ref[...]

    y1_ref[...] = raw.reshape(1, d, p2).astype(y1_ref.dtype)
    ys_ref[...] = ys.reshape(1, d, p2).astype(ys_ref.dtype)
    s1_ref[...] = s
    q1_ref[...] = q


def _stage2(y1_ref, sc1_ref, sh1_ref, gm_ref, w2_ref, b2_ref, mv_ref,
            y2_ref, s2_ref, q2_ref, slab2_ref, *, stride, margin, p2):
    d = y1_ref.shape[1]
    m = margin
    slab_ref = slab2_ref.at[pl.program_id(0) % 2]

    slab_ref[:, 0:m] = jnp.zeros((d, m), jnp.bfloat16)
    slab_ref[:, m + p2:m + p2 + m] = jnp.zeros((d, m), jnp.bfloat16)

    act = _gelu_exact(y1_ref[...].reshape(d, p2).astype(jnp.float32)
                      * sc1_ref[...] + sh1_ref[...])
    slab_ref[:, m:m + p2] = act.astype(jnp.bfloat16) * gm_ref[...]

    raw, s, q = _conv_on_slab(slab_ref, w2_ref, b2_ref[...], mv_ref[...],
                              stride=stride, margin=m, p2=p2, nrow=d)

    y2_ref[...] = raw.reshape(1, d, p2).astype(y2_ref.dtype)
    s2_ref[...] = s
    q2_ref[...] = q


def _stage3(y2_ref, sc2_ref, sh2_ref, ys_ref, out_ref,
            *, stride, p2, nrows, width):
    d = y2_ref.shape[1]
    act = _gelu_exact(y2_ref[...].reshape(d, p2).astype(jnp.float32)
                      * sc2_ref[...] + sh2_ref[...])
    v = act + ys_ref[...].reshape(d, p2).astype(jnp.float32)
    for r in range(nrows):
        out_ref[0, :, r * width:(r + 1) * width] = (
            v[:, r * stride:r * stride + width])


def _finalize_bn(s, q, gamma, beta, count):
    tot = jnp.sum(s[:, :, 0], axis=0)
    totsq = jnp.sum(q[:, :, 0], axis=0)
    mu = tot / count
    var = totsq / count - mu * mu
    inv = lax.rsqrt(jnp.maximum(var, 0.0) + _EPS)
    sc = gamma * inv
    sh = beta - mu * sc
    d = sc.shape[0]
    return sc.reshape(d, 1), sh.reshape(d, 1)


def _params(sems):
    return pltpu.CompilerParams(dimension_semantics=sems,
                                vmem_limit_bytes=100 * 1024 * 1024)


def kernel(x, skip, w1, b1, g1, be1, w2, b2, g2, be2, wsx, wss, bs):
    n, c2, hh, ww = x.shape
    _, d, hgt, wid = skip.shape
    c3 = c2 + d
    p4, p = hh * ww, hgt * wid
    stride = wid + 2
    p2 = hgt * stride
    m = max(128, pl.cdiv(stride + 1, 128) * 128)
    slen = 2 * m + p2
    bf16, f32 = jnp.bfloat16, jnp.float32

    xf = x.reshape(n, c2, p4)
    sf = skip.reshape(n, d, p)
    w1b = w1.astype(bf16)
    w2b = w2.astype(bf16)
    wsb = jnp.concatenate([wsx, wss], axis=1).astype(bf16)

    rr = np.arange(p2) // stride
    cc = np.arange(p2) % stride
    interior = cc < wid
    src = np.where(interior, (rr // 2) * ww + np.minimum(cc, wid - 1) // 2, -1)
    mup = jnp.asarray(np.arange(p4)[:, None] == src[None, :], bf16)
    maskv = jnp.asarray(interior[:, None], f32)
    gmask = jnp.asarray(interior[None, :], bf16)

    y1, s1, q1, ys = pl.pallas_call(
        partial(_stage1, stride=stride, margin=m, p2=p2, nrows=hgt,
                width=wid),
        grid=(n,),
        in_specs=[
            pl.BlockSpec((1, c2, p4), lambda i: (i, 0, 0)),
            pl.BlockSpec((1, d, p), lambda i: (i, 0, 0)),
            pl.BlockSpec((p4, p2), lambda i: (0, 0)),
            pl.BlockSpec((9, d, c3), lambda i: (0, 0, 0)),
            pl.BlockSpec((d, 1), lambda i: (0, 0)),
            pl.BlockSpec((d, c3), lambda i: (0, 0)),
            pl.BlockSpec((d, 1), lambda i: (0, 0)),
            pl.BlockSpec((p2, 1), lambda i: (0, 0)),
        ],
        out_specs=(
            pl.BlockSpec((1, d, p2), lambda i: (i, 0, 0)),
            pl.BlockSpec((1, d, 1), lambda i: (i, 0, 0)),
            pl.BlockSpec((1, d, 1), lambda i: (i, 0, 0)),
            pl.BlockSpec((1, d, p2), lambda i: (i, 0, 0)),
        ),
        out_shape=(
            jax.ShapeDtypeStruct((n, d, p2), bf16),
            jax.ShapeDtypeStruct((n, d, 1), f32),
            jax.ShapeDtypeStruct((n, d, 1), f32),
            jax.ShapeDtypeStruct((n, d, p2), bf16),
        ),
        scratch_shapes=[pltpu.VMEM((2, c3, slen), bf16)],
        compiler_params=_params(("parallel",)),
    )(xf, sf, mup, w1b, b1, wsb, bs, maskv)

    sc1, sh1 = _finalize_bn(s1, q1, g1, be1, float(n * p))

    y2, s2, q2 = pl.pallas_call(
        partial(_stage2, stride=stride, margin=m, p2=p2),
        grid=(n,),
        in_specs=[
            pl.BlockSpec((1, d, p2), lambda i: (i, 0, 0)),
            pl.BlockSpec((d, 1), lambda i: (0, 0)),
            pl.BlockSpec((d, 1), lambda i: (0, 0)),
            pl.BlockSpec((1, p2), lambda i: (0, 0)),
            pl.BlockSpec((9, d, d), lambda i: (0, 0, 0)),
            pl.BlockSpec((d, 1), lambda i: (0, 0)),
            pl.BlockSpec((p2, 1), lambda i: (0, 0)),
        ],
        out_specs=(
            pl.BlockSpec((1, d, p2), lambda i: (i, 0, 0)),
            pl.BlockSpec((1, d, 1), lambda i: (i, 0, 0)),
            pl.BlockSpec((1, d, 1), lambda i: (i, 0, 0)),
        ),
        out_shape=(
            jax.ShapeDtypeStruct((n, d, p2), bf16),
            jax.ShapeDtypeStruct((n, d, 1), f32),
            jax.ShapeDtypeStruct((n, d, 1), f32),
        ),
        scratch_shapes=[pltpu.VMEM((2, d, slen), bf16)],
        compiler_params=_params(("parallel",)),
    )(y1, sc1, sh1, gmask, w2b, b2, maskv)

    sc2, sh2 = _finalize_bn(s2, q2, g2, be2, float(n * p))

    out = pl.pallas_call(
        partial(_stage3, stride=stride, p2=p2, nrows=hgt, width=wid),
        grid=(n,),
        in_specs=[
            pl.BlockSpec((1, d, p2), lambda i: (i, 0, 0)),
            pl.BlockSpec((d, 1), lambda i: (0, 0)),
            pl.BlockSpec((d, 1), lambda i: (0, 0)),
            pl.BlockSpec((1, d, p2), lambda i: (i, 0, 0)),
        ],
        out_specs=pl.BlockSpec((1, d, p), lambda i: (i, 0, 0)),
        out_shape=jax.ShapeDtypeStruct((n, d, p), f32),
        compiler_params=_params(("parallel",)),
    )(y2, sc2, sh2, ys)

    return out.reshape(n, d, hgt, wid)
